# baseline (speedup 1.0000x reference)
import functools

import jax
import jax.numpy as jnp
from jax import lax
from jax.experimental import pallas as pl
from jax.experimental.pallas import tpu as pltpu
from jax.experimental.pallas import tpu_sc as plsc

NC = 2
NS = 16
L = 16
NW = NC * NS

N_ATOMS = 100000
N_BONDS = 1600000
BT = N_BONDS // NW
CHUNK = 2000
N_CHUNKS = BT // CHUNK
GROUPS = CHUNK // L

_i32 = jnp.int32
_f32 = jnp.float32

_GD = lax.GatherDimensionNumbers(
    offset_dims=(), collapsed_slice_dims=(0,), start_index_map=(0,))


def _lane_perm(x, idx):
    return lax.gather(x, idx[:, None], _GD, (1,),
                      mode=lax.GatherScatterMode.PROMISE_IN_BOUNDS)


def _sc_body(xs_hbm, ys_hbm, zs_hbm, bidx_hbm, b0_hbm, k_hbm, out_hbm,
             xs_sh, ys_sh, zs_sh, partials_sh, bufs, red_v, acc_v, sems):
    cid = lax.axis_index("c")
    sid = lax.axis_index("s")
    wid = cid * NS + sid

    @pl.when(sid == 0)
    def _():
        pltpu.sync_copy(xs_hbm, xs_sh)
        pltpu.sync_copy(ys_hbm, ys_sh)
        pltpu.sync_copy(zs_hbm, zs_sh)

    plsc.subcore_barrier()

    def issue_lin(t):
        p = t % 2
        bi_v, b0_v, k_v = bufs[p][:3]
        base = wid * BT + t * CHUNK
        sem = sems[p]
        return [
            pltpu.async_copy(
                bidx_hbm.at[pl.ds(base * 2, CHUNK * 2)], bi_v, sem),
            pltpu.async_copy(b0_hbm.at[pl.ds(base, CHUNK)], b0_v, sem),
            pltpu.async_copy(k_hbm.at[pl.ds(base, CHUNK)], k_v, sem),
        ]

    def issue_gather(t):
        p = t % 2
        bi_v = bufs[p][0]
        xg_v, yg_v, zg_v = bufs[p][3:]
        sem = sems[2 + p]
        h = CHUNK
        return [
            pltpu.async_copy(xs_sh.at[bi_v.at[pl.ds(0, h)]],
                             xg_v.at[pl.ds(0, h)], sem),
            pltpu.async_copy(ys_sh.at[bi_v.at[pl.ds(0, h)]],
                             yg_v.at[pl.ds(0, h)], sem),
            pltpu.async_copy(zs_sh.at[bi_v.at[pl.ds(0, h)]],
                             zg_v.at[pl.ds(0, h)], sem),
            pltpu.async_copy(xs_sh.at[bi_v.at[pl.ds(h, h)]],
                             xg_v.at[pl.ds(h, h)], sem),
            pltpu.async_copy(ys_sh.at[bi_v.at[pl.ds(h, h)]],
                             yg_v.at[pl.ds(h, h)], sem),
            pltpu.async_copy(zs_sh.at[bi_v.at[pl.ds(h, h)]],
                             zg_v.at[pl.ds(h, h)], sem),
        ]

    def compute(t, acc):
        p = t % 2
        b0_v, k_v = bufs[p][1], bufs[p][2]
        xg_v, yg_v, zg_v = bufs[p][3:]
        lanes = lax.iota(_i32, L)
        swp = lax.bitwise_xor(lanes, 1)
        cmp_idx = lax.bitwise_and(lanes * 2, L - 1)
        lo = lanes < (L // 2)

        def pair_r2(sl):
            xa = xg_v[sl]
            ya = yg_v[sl]
            za = zg_v[sl]
            dx = xa - _lane_perm(xa, swp)
            dy = ya - _lane_perm(ya, swp)
            dz = za - _lane_perm(za, swp)
            return dx * dx + dy * dy + dz * dz

        def group_step(g, acc_in):
            r2a = pair_r2(pl.ds(g * 2 * L, L))
            r2b = pair_r2(pl.ds(g * 2 * L + L, L))
            r2 = jnp.where(
                lo,
                _lane_perm(r2a, cmp_idx),
                _lane_perm(r2b, cmp_idx))
            r2c = jnp.maximum(r2, 1e-30)
            seed = jnp.full((L,), 0x5F3759DF, _i32) - lax.shift_right_logical(
                plsc.bitcast(r2c, _i32), 1)
            y = plsc.bitcast(seed, _f32)
            h = 0.5 * r2c
            y = y * (1.5 - h * y * y)
            y = y * (1.5 - h * y * y)
            y = y * (1.5 - h * y * y)
            r = r2 * y
            sl = pl.ds(g * L, L)
            d = r - b0_v[sl]
            return acc_in + d * d * k_v[sl] * 0.5

        return lax.fori_loop(0, GROUPS, group_step, acc, unroll=5)

    acc = jnp.zeros((L,), _f32)
    lin = {0: issue_lin(0)}
    for c in lin[0]:
        c.wait()
    gat = {0: issue_gather(0)}
    lin[1] = issue_lin(1)
    for t in range(N_CHUNKS):
        if t + 1 < N_CHUNKS:
            for c in lin.pop(t + 1):
                c.wait()
            gat[t + 1] = issue_gather(t + 1)
        for c in gat.pop(t):
            c.wait()
        acc = compute(t, acc)
        if t + 2 < N_CHUNKS:
            lin[t + 2] = issue_lin(t + 2)

    acc_v[...] = acc
    pltpu.sync_copy(acc_v, partials_sh.at[sid])
    plsc.subcore_barrier()

    @pl.when(sid == 0)
    def _():
        pltpu.sync_copy(partials_sh, red_v)
        tot = jnp.zeros((L,), _f32)
        for s in range(NS):
            tot = tot + red_v[s, :]
        acc_v[...] = tot
        pltpu.sync_copy(acc_v, out_hbm.at[cid])


@functools.partial(
    pl.kernel,
    out_type=jax.ShapeDtypeStruct((NC, L), _f32),
    mesh=plsc.VectorSubcoreMesh(
        core_axis_name="c", subcore_axis_name="s",
        num_cores=NC, num_subcores=NS),
    scratch_types=dict(
        xs_sh=pltpu.VMEM_SHARED((N_ATOMS,), _f32),
        ys_sh=pltpu.VMEM_SHARED((N_ATOMS,), _f32),
        zs_sh=pltpu.VMEM_SHARED((N_ATOMS,), _f32),
        partials_sh=pltpu.VMEM_SHARED((NS, L), _f32),
        bufs=[[pltpu.VMEM((2 * CHUNK,), _i32),
               pltpu.VMEM((CHUNK,), _f32), pltpu.VMEM((CHUNK,), _f32),
               pltpu.VMEM((2 * CHUNK,), _f32), pltpu.VMEM((2 * CHUNK,), _f32),
               pltpu.VMEM((2 * CHUNK,), _f32)] for _ in range(2)],
        red_v=pltpu.VMEM((NS, L), _f32),
        acc_v=pltpu.VMEM((L,), _f32),
        sems=[pltpu.SemaphoreType.DMA for _ in range(4)],
    ),
    compiler_params=pltpu.CompilerParams(
        needs_layout_passes=False, use_tc_tiling_on_sc=False),
)
def _harmonic_bond_sc(xs_hbm, ys_hbm, zs_hbm, bidx_hbm, b0_hbm, k_hbm,
                      out_hbm, **scr):
    _sc_body(xs_hbm, ys_hbm, zs_hbm, bidx_hbm, b0_hbm, k_hbm, out_hbm,
             scr["xs_sh"], scr["ys_sh"], scr["zs_sh"], scr["partials_sh"],
             scr["bufs"], scr["red_v"], scr["acc_v"], scr["sems"])


def kernel(coords, bonds, b0, k):
    coords = coords.astype(_f32)
    bidx = bonds.astype(_i32).reshape(-1)
    partials = _harmonic_bond_sc(
        coords[:, 0], coords[:, 1], coords[:, 2], bidx, b0, k)
    return jnp.sum(partials)


# speedup vs baseline: 11.6342x; 11.6342x over previous
import functools

import jax
import jax.numpy as jnp
from jax import lax
from jax.experimental import pallas as pl
from jax.experimental.pallas import tpu as pltpu
from jax.experimental.pallas import tpu_sc as plsc

NC = 2
NS = 16
L = 16
NW = NC * NS

N_ATOMS = 100000
N_BONDS = 1600000
BT = N_BONDS // NW
CHUNK = 2000
N_CHUNKS = BT // CHUNK
GROUPS = CHUNK // L

_i32 = jnp.int32
_f32 = jnp.float32


def _sc_body(xs_hbm, ys_hbm, zs_hbm, i0_hbm, i1_hbm, b0_hbm, k_hbm, out_hbm,
             xs_sh, ys_sh, zs_sh, partials_sh, bufs, red_v, acc_v, sems):
    cid = lax.axis_index("c")
    sid = lax.axis_index("s")
    wid = cid * NS + sid

    @pl.when(sid == 0)
    def _():
        pltpu.sync_copy(xs_hbm, xs_sh)
        pltpu.sync_copy(ys_hbm, ys_sh)
        pltpu.sync_copy(zs_hbm, zs_sh)

    plsc.subcore_barrier()

    def issue_lin(t):
        p = t % 2
        i0_v, i1_v, b0_v, k_v = bufs[p][:4]
        base = wid * BT + t * CHUNK
        sem = sems[p]
        return [
            pltpu.async_copy(i0_hbm.at[pl.ds(base, CHUNK)], i0_v, sem),
            pltpu.async_copy(i1_hbm.at[pl.ds(base, CHUNK)], i1_v, sem),
            pltpu.async_copy(b0_hbm.at[pl.ds(base, CHUNK)], b0_v, sem),
            pltpu.async_copy(k_hbm.at[pl.ds(base, CHUNK)], k_v, sem),
        ]

    def issue_gather(t):
        p = t % 2
        i0_v, i1_v = bufs[p][0], bufs[p][1]
        x0_v, y0_v, z0_v, x1_v, y1_v, z1_v = bufs[p][4:]
        sem = sems[2 + p]
        return [
            pltpu.async_copy(xs_sh.at[i0_v], x0_v, sem),
            pltpu.async_copy(ys_sh.at[i0_v], y0_v, sem),
            pltpu.async_copy(zs_sh.at[i0_v], z0_v, sem),
            pltpu.async_copy(xs_sh.at[i1_v], x1_v, sem),
            pltpu.async_copy(ys_sh.at[i1_v], y1_v, sem),
            pltpu.async_copy(zs_sh.at[i1_v], z1_v, sem),
        ]

    def compute(t, acc):
        p = t % 2
        b0_v, k_v = bufs[p][2], bufs[p][3]
        x0_v, y0_v, z0_v, x1_v, y1_v, z1_v = bufs[p][4:]

        def group_step(g, acc_in):
            sl = pl.ds(g * L, L)
            dx = x0_v[sl] - x1_v[sl]
            dy = y0_v[sl] - y1_v[sl]
            dz = z0_v[sl] - z1_v[sl]
            r2 = dx * dx + dy * dy + dz * dz
            r2c = jnp.maximum(r2, 1e-30)
            seed = jnp.full((L,), 0x5F3759DF, _i32) - lax.shift_right_logical(
                plsc.bitcast(r2c, _i32), 1)
            y = plsc.bitcast(seed, _f32)
            h = 0.5 * r2c
            y = y * (1.5 - h * y * y)
            y = y * (1.5 - h * y * y)
            y = y * (1.5 - h * y * y)
            r = r2 * y
            d = r - b0_v[sl]
            return acc_in + d * d * k_v[sl] * 0.5

        return lax.fori_loop(0, GROUPS, group_step, acc, unroll=5)

    acc = jnp.zeros((L,), _f32)
    lin = {0: issue_lin(0)}
    for c in lin[0]:
        c.wait()
    gat = {0: issue_gather(0)}
    lin[1] = issue_lin(1)
    for t in range(N_CHUNKS):
        if t + 1 < N_CHUNKS:
            for c in lin.pop(t + 1):
                c.wait()
            gat[t + 1] = issue_gather(t + 1)
        for c in gat.pop(t):
            c.wait()
        acc = compute(t, acc)
        if t + 2 < N_CHUNKS:
            lin[t + 2] = issue_lin(t + 2)

    acc_v[...] = acc
    pltpu.sync_copy(acc_v, partials_sh.at[sid])
    plsc.subcore_barrier()

    @pl.when(sid == 0)
    def _():
        pltpu.sync_copy(partials_sh, red_v)
        tot = jnp.zeros((L,), _f32)
        for s in range(NS):
            tot = tot + red_v[s, :]
        acc_v[...] = tot
        pltpu.sync_copy(acc_v, out_hbm.at[cid])


@functools.partial(
    pl.kernel,
    out_type=jax.ShapeDtypeStruct((NC, L), _f32),
    mesh=plsc.VectorSubcoreMesh(
        core_axis_name="c", subcore_axis_name="s",
        num_cores=NC, num_subcores=NS),
    scratch_types=dict(
        xs_sh=pltpu.VMEM_SHARED((N_ATOMS,), _f32),
        ys_sh=pltpu.VMEM_SHARED((N_ATOMS,), _f32),
        zs_sh=pltpu.VMEM_SHARED((N_ATOMS,), _f32),
        partials_sh=pltpu.VMEM_SHARED((NS, L), _f32),
        bufs=[[pltpu.VMEM((CHUNK,), _i32), pltpu.VMEM((CHUNK,), _i32),
               pltpu.VMEM((CHUNK,), _f32), pltpu.VMEM((CHUNK,), _f32),
               pltpu.VMEM((CHUNK,), _f32), pltpu.VMEM((CHUNK,), _f32),
               pltpu.VMEM((CHUNK,), _f32), pltpu.VMEM((CHUNK,), _f32),
               pltpu.VMEM((CHUNK,), _f32),
               pltpu.VMEM((CHUNK,), _f32)] for _ in range(2)],
        red_v=pltpu.VMEM((NS, L), _f32),
        acc_v=pltpu.VMEM((L,), _f32),
        sems=[pltpu.SemaphoreType.DMA for _ in range(4)],
    ),
    compiler_params=pltpu.CompilerParams(
        needs_layout_passes=False, use_tc_tiling_on_sc=False),
)
def _harmonic_bond_sc(xs_hbm, ys_hbm, zs_hbm, i0_hbm, i1_hbm, b0_hbm, k_hbm,
                      out_hbm, **scr):
    _sc_body(xs_hbm, ys_hbm, zs_hbm, i0_hbm, i1_hbm, b0_hbm, k_hbm, out_hbm,
             scr["xs_sh"], scr["ys_sh"], scr["zs_sh"], scr["partials_sh"],
             scr["bufs"], scr["red_v"], scr["acc_v"], scr["sems"])


def kernel(coords, bonds, b0, k):
    coords = coords.astype(_f32)
    bonds = bonds.astype(_i32)
    partials = _harmonic_bond_sc(
        coords[:, 0], coords[:, 1], coords[:, 2],
        bonds[:, 0], bonds[:, 1], b0, k)
    return jnp.sum(partials)
